# baseline (speedup 1.0000x reference)
import functools

import jax
import jax.numpy as jnp
from jax import lax
from jax.experimental import pallas as pl
from jax.experimental.pallas import tpu as pltpu
from jax.experimental.pallas import tpu_sc as plsc

B = 16384
F = 26
W = 32
GW = 128
NC = 2
NS = 16
NW = NC * NS
BPW = B // NW
NB = BPW // GW
NSTEP = F * NB
L = 16

_mesh = plsc.VectorSubcoreMesh(core_axis_name="c", subcore_axis_name="s")


@functools.partial(
    pl.kernel,
    mesh=_mesh,
    compiler_params=pltpu.CompilerParams(
        use_tc_tiling_on_sc=True, needs_layout_passes=False
    ),
    out_type=jax.ShapeDtypeStruct((B, GW), jnp.float32),
    scratch_types=[
        pltpu.VMEM((NSTEP, GW), jnp.int32),
        pltpu.VMEM((2, GW), jnp.int32),
        pltpu.VMEM((2, GW, GW), jnp.float32),
        pltpu.VMEM((NB, GW, GW), jnp.float32),
        pltpu.SemaphoreType.DMA,
        pltpu.SemaphoreType.DMA,
        pltpu.SemaphoreType.DMA,
    ],
)
def _embed_sum(x_hbm, emb_hbm, out_hbm, xraw_v, rowbuf_v, rows_v, acc_v,
               sem_a, sem_b, sem_o):
    wid = lax.axis_index("c") * NS + lax.axis_index("s")

    pltpu.sync_copy(x_hbm.at[wid], xraw_v)

    sems = (sem_a, sem_b)
    iota = lax.iota(jnp.int32, L)
    zeros = jnp.zeros((L,), jnp.float32)

    def zbody(j, carry):
        for blk in range(NB):
            acc_v[blk, j, pl.ds(0, L)] = zeros
            acc_v[blk, j, pl.ds(L, L)] = zeros
        return carry

    lax.fori_loop(0, GW, zbody, 0)

    def issue(step, buf):
        for k in range(GW // L):
            rowbuf_v[buf, pl.ds(k * L, L)] = lax.shift_right_logical(
                xraw_v[step, pl.ds(k * L, L)], 2
            )
        pltpu.async_copy(emb_hbm.at[rowbuf_v.at[buf]], rows_v.at[buf], sems[buf])

    def wait(buf):
        pltpu.make_async_copy(
            emb_hbm.at[pl.ds(0, GW)], rows_v.at[buf], sems[buf]
        ).wait()

    def consume(step, buf):
        blkv = iota * 0 + (step & (NB - 1))
        bufv = iota * 0 + buf
        for lg in range(GW // L):
            elv = lg * L + iota
            cb = (xraw_v[step, pl.ds(lg * L, L)] & 3) * W
            for c in range(W):
                v = plsc.load_gather(rows_v, [bufv, elv, cb + c])
                plsc.addupdate_scatter(acc_v, [blkv, elv, iota * 0 + c], v)

    issue(0, 0)

    def body(i, carry):
        s = 2 * i
        issue(s + 1, 1)
        wait(0)
        consume(s, 0)

        @pl.when(s + 2 < NSTEP)
        def _():
            issue(s + 2, 0)

        wait(1)
        consume(s + 1, 1)
        return carry

    lax.fori_loop(0, NSTEP // 2, body, 0)

    base = wid * BPW
    for blk in range(NB):
        pltpu.async_copy(
            acc_v.at[blk], out_hbm.at[pl.ds(base + blk * GW, GW)], sem_o
        )
    for blk in range(NB):
        pltpu.make_async_copy(
            emb_hbm.at[pl.ds(0, GW)], acc_v.at[blk], sem_o
        ).wait()


def kernel(x, embeddings):
    xw = (
        x.astype(jnp.int32)
        .T.reshape(F, NW, BPW)
        .swapaxes(0, 1)
        .reshape(NW, NSTEP, GW)
    )
    emb = embeddings.reshape(-1, GW)
    return _embed_sum(xw, emb)[:, :W]


# speedup vs baseline: 1.6387x; 1.6387x over previous
import functools

import jax
import jax.numpy as jnp
from jax import lax
from jax.experimental import pallas as pl
from jax.experimental.pallas import tpu as pltpu
from jax.experimental.pallas import tpu_sc as plsc

B = 16384
F = 26
W = 32
NC = 2
NS = 16
NW = NC * NS
BPW = B // NW
C = 16
IPC = C * F
NCHUNK = BPW // C
HALF = W // 2

_mesh = plsc.VectorSubcoreMesh(core_axis_name="c", subcore_axis_name="s")


@functools.partial(
    pl.kernel,
    mesh=_mesh,
    compiler_params=pltpu.CompilerParams(
        use_tc_tiling_on_sc=False, needs_layout_passes=False
    ),
    out_type=jax.ShapeDtypeStruct((B, W), jnp.float32),
    scratch_types=[
        pltpu.VMEM((NCHUNK, IPC), jnp.int32),
        pltpu.VMEM((2, IPC, W), jnp.bfloat16),
        pltpu.VMEM((BPW, W), jnp.float32),
        pltpu.SemaphoreType.DMA,
        pltpu.SemaphoreType.DMA,
    ],
)
def _embed_sum(x_hbm, emb_hbm, out_hbm, idx_v, rows_v, out_v, sem_a, sem_b):
    wid = lax.axis_index("c") * NS + lax.axis_index("s")

    pltpu.sync_copy(x_hbm.at[wid], idx_v)

    sems = (sem_a, sem_b)

    def issue(chunk, buf):
        pltpu.async_copy(emb_hbm.at[idx_v.at[chunk]], rows_v.at[buf], sems[buf])

    def wait(buf):
        pltpu.make_async_copy(
            emb_hbm.at[pl.ds(0, IPC)], rows_v.at[buf], sems[buf]
        ).wait()

    iota = lax.iota(jnp.int32, HALF)

    def row_halves(buf, r):
        return plsc.unpack(rows_v[buf, r], format=plsc.PackFormat.INTERLEAVED)

    def compute(chunk, buf):
        for e in range(C):
            r0 = e * F
            acc_ev, acc_od = row_halves(buf, r0)
            for f in range(1, F):
                ev, od = row_halves(buf, r0 + f)
                acc_ev = acc_ev + ev
                acc_od = acc_od + od
            rowv = iota * 0 + (chunk * C + e)
            plsc.store_scatter(out_v, [rowv, iota * 2], acc_ev)
            plsc.store_scatter(out_v, [rowv, iota * 2 + 1], acc_od)

    issue(0, 0)

    def body(i, carry):
        g = 2 * i
        issue(g + 1, 1)
        wait(0)
        compute(g, 0)

        @pl.when(g + 2 < NCHUNK)
        def _():
            issue(g + 2, 0)

        wait(1)
        compute(g + 1, 1)
        return carry

    lax.fori_loop(0, NCHUNK // 2, body, 0)

    pltpu.sync_copy(out_v, out_hbm.at[pl.ds(wid * BPW, BPW)])


def kernel(x, embeddings):
    x = x.astype(jnp.int32).reshape(NW, NCHUNK, IPC)
    emb = embeddings.astype(jnp.bfloat16)
    return _embed_sum(x, emb)


# speedup vs baseline: 1.9255x; 1.1750x over previous
import functools

import jax
import jax.numpy as jnp
from jax import lax
from jax.experimental import pallas as pl
from jax.experimental.pallas import tpu as pltpu
from jax.experimental.pallas import tpu_sc as plsc

B = 16384
F = 26
W = 32
NC = 2
NS = 16
NW = NC * NS
BPW = B // NW
C = 16
IPC = C * F
NCHUNK = BPW // C
HALF = W // 2

_mesh = plsc.VectorSubcoreMesh(core_axis_name="c", subcore_axis_name="s")


@functools.partial(
    pl.kernel,
    mesh=_mesh,
    compiler_params=pltpu.CompilerParams(use_tc_tiling_on_sc=False),
    out_type=jax.ShapeDtypeStruct((B, W), jnp.float32),
    scratch_types=[
        pltpu.VMEM((NCHUNK, IPC), jnp.int32),
        pltpu.VMEM((2, IPC, W), jnp.float32),
        pltpu.VMEM((BPW, W), jnp.float32),
        pltpu.SemaphoreType.DMA,
        pltpu.SemaphoreType.DMA,
    ],
)
def _embed_sum(x_hbm, emb_hbm, out_hbm, idx_v, rows_v, out_v, sem_a, sem_b):
    wid = lax.axis_index("c") * NS + lax.axis_index("s")

    pltpu.sync_copy(x_hbm.at[wid], idx_v)

    sems = (sem_a, sem_b)

    def issue(chunk, buf):
        pltpu.async_copy(emb_hbm.at[idx_v.at[chunk]], rows_v.at[buf], sems[buf])

    def wait(buf):
        pltpu.make_async_copy(
            emb_hbm.at[pl.ds(0, IPC)], rows_v.at[buf], sems[buf]
        ).wait()

    def compute(chunk, buf):
        for e in range(C):
            r0 = e * F
            acc_lo = rows_v[buf, r0, pl.ds(0, HALF)]
            acc_hi = rows_v[buf, r0, pl.ds(HALF, HALF)]
            for f in range(1, F):
                acc_lo = acc_lo + rows_v[buf, r0 + f, pl.ds(0, HALF)]
                acc_hi = acc_hi + rows_v[buf, r0 + f, pl.ds(HALF, HALF)]
            row = chunk * C + e
            out_v[row, pl.ds(0, HALF)] = acc_lo
            out_v[row, pl.ds(HALF, HALF)] = acc_hi

    issue(0, 0)

    def body(i, carry):
        g = 2 * i
        issue(g + 1, 1)
        wait(0)
        compute(g, 0)

        @pl.when(g + 2 < NCHUNK)
        def _():
            issue(g + 2, 0)

        wait(1)
        compute(g + 1, 1)
        return carry

    lax.fori_loop(0, NCHUNK // 2, body, 0)

    pltpu.sync_copy(out_v, out_hbm.at[pl.ds(wid * BPW, BPW)])


def kernel(x, embeddings):
    x = x.astype(jnp.int32).reshape(NW, NCHUNK, IPC)
    return _embed_sum(x, embeddings)


# speedup vs baseline: 1.9287x; 1.0017x over previous
import functools

import jax
import jax.numpy as jnp
from jax import lax
from jax.experimental import pallas as pl
from jax.experimental.pallas import tpu as pltpu
from jax.experimental.pallas import tpu_sc as plsc

B = 16384
F = 26
W = 32
NC = 2
NS = 16
NW = NC * NS
BPW = B // NW
C = 16
IPC = C * F
NCHUNK = BPW // C
HALF = W // 2

def _embed_sum(x_hbm, emb_hbm, out_hbm, idx_v, rows_v, out_v, sem_a, sem_b):
    wid = lax.axis_index("c") * NS + lax.axis_index("s")

    pltpu.sync_copy(x_hbm.at[wid], idx_v)

    sems = (sem_a, sem_b)

    def issue(chunk, buf):
        pltpu.async_copy(emb_hbm.at[idx_v.at[chunk]], rows_v.at[buf], sems[buf])

    def wait(buf):
        pltpu.make_async_copy(
            emb_hbm.at[pl.ds(0, IPC)], rows_v.at[buf], sems[buf]
        ).wait()

    def compute(chunk, buf):
        for e in range(C):
            r0 = e * F
            acc_lo = rows_v[buf, r0, pl.ds(0, HALF)]
            acc_hi = rows_v[buf, r0, pl.ds(HALF, HALF)]
            for f in range(1, F):
                acc_lo = acc_lo + rows_v[buf, r0 + f, pl.ds(0, HALF)]
                acc_hi = acc_hi + rows_v[buf, r0 + f, pl.ds(HALF, HALF)]
            row = chunk * C + e
            out_v[row, pl.ds(0, HALF)] = acc_lo
            out_v[row, pl.ds(HALF, HALF)] = acc_hi

    issue(0, 0)

    def body(i, carry):
        g = 2 * i
        issue(g + 1, 1)
        wait(0)
        compute(g, 0)

        @pl.when(g + 2 < NCHUNK)
        def _():
            issue(g + 2, 0)

        wait(1)
        compute(g + 1, 1)
        return carry

    lax.fori_loop(0, NCHUNK // 2, body, 0)

    pltpu.sync_copy(out_v, out_hbm.at[pl.ds(wid * BPW, BPW)])


@functools.cache
def _build():
    mesh = plsc.VectorSubcoreMesh(core_axis_name="c", subcore_axis_name="s")
    return pl.kernel(
        _embed_sum,
        mesh=mesh,
        compiler_params=pltpu.CompilerParams(use_tc_tiling_on_sc=False),
        out_type=jax.ShapeDtypeStruct((B, W), jnp.float32),
        scratch_types=[
            pltpu.VMEM((NCHUNK, IPC), jnp.int32),
            pltpu.VMEM((2, IPC, W), jnp.float32),
            pltpu.VMEM((BPW, W), jnp.float32),
            pltpu.SemaphoreType.DMA,
            pltpu.SemaphoreType.DMA,
        ],
    )


def kernel(x, embeddings):
    x = x.astype(jnp.int32).reshape(NW, NCHUNK, IPC)
    return _build()(x, embeddings)
